# Initial kernel scaffold; baseline (speedup 1.0000x reference)
import jax
import jax.numpy as jnp
from jax.experimental import pallas as pl


def kernel(vertices, faces):
    raise NotImplementedError("write your pallas kernel here")


# speedup vs baseline: 100.4193x; 100.4193x over previous
import functools

import jax
import jax.numpy as jnp
from jax import lax
from jax.experimental import pallas as pl
from jax.experimental.pallas import tpu as pltpu
from jax.experimental.pallas import tpu_sc as plsc

B = 4
V = 100000
F = 200000
W = 16
NC = 2
NS = 16
NW = NC * NS
K = 128
CHUNKS = 49
FT = K * CHUNKS
F_PAD = NW * FT
V_PAD = 100352
RPS = V_PAD // NS
ZR = 784

_mesh = plsc.VectorSubcoreMesh(core_axis_name="c", subcore_axis_name="s")


@functools.partial(
    pl.kernel,
    out_type=jax.ShapeDtypeStruct((NC, V_PAD, W), jnp.float32),
    mesh=_mesh,
    scratch_types=[
        pltpu.VMEM_SHARED((V_PAD, W), jnp.float32),
        pltpu.VMEM((ZR, W), jnp.float32),
        pltpu.VMEM((K,), jnp.int32),
        pltpu.VMEM((K,), jnp.int32),
        pltpu.VMEM((K,), jnp.int32),
        pltpu.VMEM((K, W), jnp.float32),
        pltpu.VMEM((K, W), jnp.float32),
        pltpu.VMEM((K, W), jnp.float32),
        pltpu.VMEM((K, W), jnp.float32),
        pltpu.VMEM((K, W), jnp.float32),
        pltpu.VMEM((K, W), jnp.float32),
        pltpu.VMEM((K, W), jnp.float32),
        pltpu.SemaphoreType.DMA,
    ],
    compiler_params=pltpu.CompilerParams(use_tc_tiling_on_sc=False),
)
def _face_scatter(rot1_hbm, rot2_hbm, f0_hbm, f1_hbm, f2_hbm, out_hbm,
                  acc, zbuf, i0, i1, i2, a0, b0, a1, b1, a2, b2, nbuf, sem):
    cid = lax.axis_index("c")
    sid = lax.axis_index("s")
    tid = cid * NS + sid
    row0 = sid * RPS

    def _zrow(r, carry):
        zbuf[r] = jnp.zeros((W,), jnp.float32)
        return carry
    lax.fori_loop(0, ZR, _zrow, 0)

    def _zacc(i, carry):
        pltpu.sync_copy(zbuf, acc.at[pl.ds(row0 + i * ZR, ZR)])
        return carry
    lax.fori_loop(0, RPS // ZR, _zacc, 0)
    plsc.subcore_barrier()

    fbase = tid * FT

    def _chunk(j, carry):
        base = fbase + j * K
        pltpu.sync_copy(f0_hbm.at[pl.ds(base, K)], i0)
        pltpu.sync_copy(f1_hbm.at[pl.ds(base, K)], i1)
        pltpu.sync_copy(f2_hbm.at[pl.ds(base, K)], i2)
        c0 = pltpu.async_copy(rot1_hbm.at[i0], a0, sem)
        c1 = pltpu.async_copy(rot2_hbm.at[i0], b0, sem)
        c2 = pltpu.async_copy(rot1_hbm.at[i1], a1, sem)
        c3 = pltpu.async_copy(rot2_hbm.at[i1], b1, sem)
        c4 = pltpu.async_copy(rot1_hbm.at[i2], a2, sem)
        c5 = pltpu.async_copy(rot2_hbm.at[i2], b2, sem)
        c0.wait(); c1.wait(); c2.wait(); c3.wait(); c4.wait(); c5.wait()

        def _face(k, carry2):
            p1 = a2[k] - a1[k]
            p2 = b2[k] - b1[k]
            q1 = a0[k] - a1[k]
            q2 = b0[k] - b1[k]
            nbuf[k] = p1 * q2 - p2 * q1
            return carry2
        lax.fori_loop(0, K, _face, 0, unroll=8)

        pltpu.sync_copy(nbuf, acc.at[i0], add=True)
        pltpu.sync_copy(nbuf, acc.at[i1], add=True)
        pltpu.sync_copy(nbuf, acc.at[i2], add=True)
        return carry
    lax.fori_loop(0, CHUNKS, _chunk, 0)

    plsc.subcore_barrier()
    pltpu.sync_copy(acc.at[pl.ds(row0, RPS)],
                    out_hbm.at[cid, pl.ds(row0, RPS)])


_RB = 1568


def _combine_body(p0_ref, p1_ref, o_ref):
    s = p0_ref[...] + p1_ref[...]
    sq = s * s
    ii = lax.broadcasted_iota(jnp.int32, (W, W), 0)
    jj = lax.broadcasted_iota(jnp.int32, (W, W), 1)
    g = ((ii // 3 == jj // 3) & (ii < B * 3) & (jj < B * 3)).astype(jnp.float32)
    gs = jnp.dot(sq, g, preferred_element_type=jnp.float32)
    o_ref[...] = s * lax.rsqrt(jnp.maximum(gs, 1e-12))


_combine = pl.pallas_call(
    _combine_body,
    out_shape=jax.ShapeDtypeStruct((V_PAD, W), jnp.float32),
    grid=(V_PAD // _RB,),
    in_specs=[
        pl.BlockSpec((_RB, W), lambda i: (i, 0)),
        pl.BlockSpec((_RB, W), lambda i: (i, 0)),
    ],
    out_specs=pl.BlockSpec((_RB, W), lambda i: (i, 0)),
)


def kernel(vertices, faces):
    vt = jnp.transpose(vertices, (1, 0, 2))
    r1 = vt[:, :, jnp.array([1, 2, 0])].reshape(V, B * 3)
    r2 = vt[:, :, jnp.array([2, 0, 1])].reshape(V, B * 3)
    pad = jnp.zeros((V, W - B * 3), jnp.float32)
    rot1 = jnp.concatenate([r1, pad], axis=1)
    rot2 = jnp.concatenate([r2, pad], axis=1)
    ft = jnp.zeros((3, F_PAD), jnp.int32).at[:, :F].set(faces.T)
    partials = _face_scatter(rot1, rot2, ft[0], ft[1], ft[2])
    normalized = _combine(partials[0], partials[1])
    return normalized[:V, :B * 3].reshape(V, B, 3).transpose(1, 0, 2)
